# baseline (speedup 1.0000x reference)
import jax
import jax.numpy as jnp
from jax.experimental import pallas as pl
from jax.experimental.pallas import tpu as pltpu

_NUM, _P, _C = 32, 8732, 21
_PB = 1096
_NBLK = 8


def _body(prior_ref,
          lm1, lv1, lp1, lm2, lv2, lp2, lm3, lv3, lp3, lm4, lv4, lp4,
          cm1, cv1, cp1, cm2, cv2, cp2, cm3, cv3, cp3, cm4, cv4, cp4,
          out_ref):
    m1, m2, m3, m4 = lm1[0], lm2[0], lm3[0], lm4[0]
    w1, w2, w3, w4 = lp1[0], lp2[0], lp3[0], lp4[0]
    s1, s2, s3, s4 = lv1[0], lv2[0], lv3[0], lv4[0]
    new_loc = w1 * m1 + w2 * m2 + w3 * m3 + w4 * m4
    al_uc = w1 * s1 + w2 * s2 + w3 * s3 + w4 * s4
    ep_uc = (w1 * (m1 - new_loc) ** 2 + w2 * (m2 - new_loc) ** 2
             + w3 * (m3 - new_loc) ** 2 + w4 * (m4 - new_loc) ** 2)

    a1, a2, a3, a4 = cm1[0], cm2[0], cm3[0], cm4[0]
    q1, q2, q3, q4 = cp1[0], cp2[0], cp3[0], cp4[0]
    t1, t2, t3, t4 = cv1[0], cv2[0], cv3[0], cv4[0]
    new_conf = q1 * a1 + q2 * a2 + q3 * a3 + q4 * a4
    cls_al = q1 * t1 + q2 * t2 + q3 * t3 + q4 * t4
    cls_ep = (q1 * (a1 - new_conf) ** 2 + q2 * (a2 - new_conf) ** 2
              + q3 * (a3 - new_conf) ** 2 + q4 * (a4 - new_conf) ** 2)

    prior = prior_ref[...]
    pxy = prior[:, :2]
    pwh = prior[:, 2:]
    cxcy = pxy + new_loc[:, :2] * 0.1 * pwh
    wh = pwh * jnp.exp(new_loc[:, 2:] * 0.2)
    x1y1 = cxcy - wh * 0.5
    x2y2 = x1y1 + wh

    out_ref[0] = jnp.concatenate(
        [x1y1, x2y2, al_uc, ep_uc, new_conf, cls_al, cls_ep], axis=-1)


def kernel(prior_data, loc_mu_1, loc_var_1, loc_pi_1, loc_mu_2, loc_var_2,
           loc_pi_2, loc_mu_3, loc_var_3, loc_pi_3, loc_mu_4, loc_var_4,
           loc_pi_4, conf_mu_1, conf_var_1, conf_pi_1, conf_mu_2, conf_var_2,
           conf_pi_2, conf_mu_3, conf_var_3, conf_pi_3, conf_mu_4, conf_var_4,
           conf_pi_4):
    loc_spec = pl.BlockSpec((1, _PB, 4), lambda n, j: (n, j, 0))
    conf_spec = pl.BlockSpec((1, _PB, _C), lambda n, j: (n, j, 0))
    prior_spec = pl.BlockSpec((_PB, 4), lambda n, j: (j, 0))
    return pl.pallas_call(
        _body,
        grid=(_NUM, _NBLK),
        in_specs=[prior_spec] + [loc_spec] * 12 + [conf_spec] * 12,
        out_specs=pl.BlockSpec((1, _PB, 75), lambda n, j: (n, j, 0)),
        out_shape=jax.ShapeDtypeStruct((_NUM, _P, 75), jnp.float32),
        compiler_params=pltpu.CompilerParams(
            dimension_semantics=("parallel", "arbitrary")),
    )(prior_data, loc_mu_1, loc_var_1, loc_pi_1, loc_mu_2, loc_var_2,
      loc_pi_2, loc_mu_3, loc_var_3, loc_pi_3, loc_mu_4, loc_var_4, loc_pi_4,
      conf_mu_1, conf_var_1, conf_pi_1, conf_mu_2, conf_var_2, conf_pi_2,
      conf_mu_3, conf_var_3, conf_pi_3, conf_mu_4, conf_var_4, conf_pi_4)
